# baseline (speedup 1.0000x reference)
import functools

import jax
import jax.numpy as jnp
from jax import lax
from jax.experimental import pallas as pl
from jax.experimental.pallas import tpu as pltpu
from jax.experimental.pallas import tpu_sc as plsc

_NC = 2
_NS = 16
_NW = _NC * _NS
_L = 16
_CHUNK = 128
_INFLIGHT = 16


@functools.lru_cache(maxsize=None)
def _lr_call(B, F, V):
    n_per_w = (B * F) // _NW
    rows_per_w = n_per_w // _CHUNK
    b_per_w = B // _NW
    groups = b_per_w // _L

    mesh = plsc.VectorSubcoreMesh(core_axis_name="c", subcore_axis_name="s")

    @functools.partial(
        pl.kernel,
        out_type=jax.ShapeDtypeStruct((B,), jnp.float32),
        mesh=mesh,
        scratch_types=[
            pltpu.VMEM((rows_per_w, _CHUNK), jnp.int32),
            pltpu.VMEM((n_per_w,), jnp.float32),
            pltpu.VMEM((b_per_w,), jnp.float32),
            pltpu.SemaphoreType.DMA,
        ],
    )
    def lr_kernel(x2_hbm, table_hbm, out_hbm, idx_v, vals_v, out_v, sem):
        wid = lax.axis_index("s") * _NC + lax.axis_index("c")
        row0 = wid * rows_per_w
        base = wid * b_per_w

        pltpu.sync_copy(x2_hbm.at[pl.ds(row0, rows_per_w)], idx_v)

        def fire(j, _):
            @pl.when(j < rows_per_w)
            def _():
                pltpu.async_copy(
                    table_hbm.at[idx_v.at[j]],
                    vals_v.at[pl.ds(j * _CHUNK, _CHUNK)],
                    sem,
                )

            @pl.when(j >= _INFLIGHT)
            def _():
                jp = j - _INFLIGHT
                pltpu.make_async_copy(
                    table_hbm.at[idx_v.at[jp]],
                    vals_v.at[pl.ds(jp * _CHUNK, _CHUNK)],
                    sem,
                ).wait()
            return 0

        lax.fori_loop(0, rows_per_w + _INFLIGHT, fire, 0, unroll=False)

        cpw = b_per_w // _CHUNK
        for g in range(groups):
            c = (g * _L) // _CHUNK
            t = (g * _L) % _CHUNK
            acc = jnp.zeros((_L,), jnp.float32)
            for k in range(F):
                acc = acc + vals_v[pl.ds((k * cpw + c) * _CHUNK + t, _L)]
            out_v[pl.ds(g * _L, _L)] = acc

        pltpu.sync_copy(out_v, out_hbm.at[pl.ds(base, b_per_w)])

    return lr_kernel


def _fm_body(x_ref, o_ref):
    x = x_ref[...]
    s = jnp.sum(x, axis=1)
    ss = jnp.sum(s * s, axis=1, keepdims=True)
    sq = jnp.sum(jnp.sum(x * x, axis=1), axis=1, keepdims=True)
    o_ref[...] = 0.5 * (ss - sq)


@functools.lru_cache(maxsize=None)
def _fm_call(B, F, D):
    BB = 512
    return pl.pallas_call(
        _fm_body,
        grid=(B // BB,),
        in_specs=[pl.BlockSpec((BB, F, D), lambda i: (i, 0, 0))],
        out_specs=pl.BlockSpec((BB, 1), lambda i: (i, 0)),
        out_shape=jax.ShapeDtypeStruct((B, 1), jnp.float32),
    )


def kernel(X, feature_emb, lr_table, bias):
    B, F = X.shape
    D = feature_emb.shape[2]
    V = lr_table.shape[0]

    bpw = B // _NW
    cpw = bpw // _CHUNK
    x2 = (X.T.reshape(F, _NW, cpw, _CHUNK)
          .transpose(1, 0, 2, 3)
          .reshape((B * F) // _CHUNK, _CHUNK))
    lr = _lr_call(B, F, V)(x2, lr_table.reshape(-1))

    fm = _fm_call(B, F, D)(feature_emb)

    return fm + lr[:, None] + bias[None, :]


# speedup vs baseline: 3.4890x; 3.4890x over previous
import functools

import jax
import jax.numpy as jnp
from jax import lax
from jax.experimental import pallas as pl
from jax.experimental.pallas import tpu as pltpu
from jax.experimental.pallas import tpu_sc as plsc

_NC = 2
_NS = 16
_NW = _NC * _NS
_L = 16
_CHUNK = 128
_INFLIGHT = 16


@functools.lru_cache(maxsize=None)
def _lr_call(B, F, V):
    n_per_w = (B * F) // _NW
    rows_per_w = n_per_w // _CHUNK
    b_per_w = B // _NW
    groups = b_per_w // _L

    mesh = plsc.VectorSubcoreMesh(core_axis_name="c", subcore_axis_name="s")

    @functools.partial(
        pl.kernel,
        out_type=jax.ShapeDtypeStruct((B,), jnp.float32),
        mesh=mesh,
        scratch_types=[
            pltpu.VMEM((rows_per_w, _CHUNK), jnp.int32),
            pltpu.VMEM((n_per_w,), jnp.float32),
            pltpu.VMEM((b_per_w,), jnp.float32),
            pltpu.SemaphoreType.DMA,
        ],
    )
    def lr_kernel(x2_hbm, table_hbm, out_hbm, idx_v, vals_v, out_v, sem):
        wid = lax.axis_index("s") * _NC + lax.axis_index("c")
        row0 = wid * rows_per_w
        base = wid * b_per_w

        pltpu.sync_copy(x2_hbm.at[pl.ds(row0, rows_per_w)], idx_v)

        def fire(j, _):
            @pl.when(j < rows_per_w)
            def _():
                pltpu.async_copy(
                    table_hbm.at[idx_v.at[j]],
                    vals_v.at[pl.ds(j * _CHUNK, _CHUNK)],
                    sem,
                )

            @pl.when(j >= _INFLIGHT)
            def _():
                jp = j - _INFLIGHT
                pltpu.make_async_copy(
                    table_hbm.at[idx_v.at[jp]],
                    vals_v.at[pl.ds(jp * _CHUNK, _CHUNK)],
                    sem,
                ).wait()
            return 0

        lax.fori_loop(0, rows_per_w + _INFLIGHT, fire, 0, unroll=False)

        cpw = b_per_w // _CHUNK
        for g in range(groups):
            c = (g * _L) // _CHUNK
            t = (g * _L) % _CHUNK
            acc = jnp.zeros((_L,), jnp.float32)
            for k in range(F):
                acc = acc + vals_v[pl.ds((k * cpw + c) * _CHUNK + t, _L)]
            out_v[pl.ds(g * _L, _L)] = acc

        pltpu.sync_copy(out_v, out_hbm.at[pl.ds(base, b_per_w)])

    return lr_kernel


@functools.lru_cache(maxsize=None)
def _fm_call(B, F, D):
    BB = 1024

    def _fm_body(x_ref, o_ref):
        x = x_ref[...]
        x3 = x.reshape(F, D, BB)
        s = jnp.sum(x3, axis=0)
        ss = jnp.sum(s * s, axis=0)
        sq = jnp.sum(x * x, axis=0)
        o_ref[...] = 0.5 * (ss - sq)

    return pl.pallas_call(
        _fm_body,
        grid=(B // BB,),
        in_specs=[pl.BlockSpec((F * D, BB), lambda i: (0, i))],
        out_specs=pl.BlockSpec((BB,), lambda i: (i,)),
        out_shape=jax.ShapeDtypeStruct((B,), jnp.float32),
    )


def kernel(X, feature_emb, lr_table, bias):
    B, F = X.shape
    D = feature_emb.shape[2]
    V = lr_table.shape[0]

    bpw = B // _NW
    cpw = bpw // _CHUNK
    x2 = (X.T.reshape(F, _NW, cpw, _CHUNK)
          .transpose(1, 0, 2, 3)
          .reshape((B * F) // _CHUNK, _CHUNK))
    lr = _lr_call(B, F, V)(x2, lr_table.T.reshape(-1))

    xT = feature_emb.reshape(B, F * D).T
    fm = _fm_call(B, F, D)(xT)

    return (fm + lr + bias[0])[:, None]


# speedup vs baseline: 6.6222x; 1.8980x over previous
import functools

import jax
import jax.numpy as jnp
from jax import lax
from jax.experimental import pallas as pl
from jax.experimental.pallas import tpu as pltpu
from jax.experimental.pallas import tpu_sc as plsc

_NC = 2
_NS = 16
_NW = _NC * _NS
_L = 16
_CHUNK = 128
_INFLIGHT = 16


@functools.lru_cache(maxsize=None)
def _lr_call(B, F, V):
    n_per_w = (B * F) // _NW
    rows_per_w = n_per_w // _CHUNK
    b_per_w = B // _NW
    groups = b_per_w // _L

    mesh = plsc.VectorSubcoreMesh(core_axis_name="c", subcore_axis_name="s")

    @functools.partial(
        pl.kernel,
        out_type=jax.ShapeDtypeStruct((B,), jnp.float32),
        mesh=mesh,
        scratch_types=[
            pltpu.VMEM((rows_per_w, 1, _CHUNK), jnp.int32),
            pltpu.VMEM((1, n_per_w), jnp.float32),
            pltpu.VMEM((b_per_w,), jnp.float32),
            pltpu.SemaphoreType.DMA,
        ],
    )
    def lr_kernel(x2_hbm, table_hbm, out_hbm, idx_v, vals_v, out_v, sem):
        wid = lax.axis_index("s") * _NC + lax.axis_index("c")
        row0 = wid * rows_per_w
        base = wid * b_per_w

        pltpu.sync_copy(x2_hbm.at[pl.ds(row0, rows_per_w)], idx_v)

        def fire(j, _):
            @pl.when(j < rows_per_w)
            def _():
                pltpu.async_copy(
                    table_hbm.at[idx_v.at[j]],
                    vals_v.at[:, pl.ds(j * _CHUNK, _CHUNK)],
                    sem,
                )

            @pl.when(j >= _INFLIGHT)
            def _():
                jp = j - _INFLIGHT
                pltpu.make_async_copy(
                    table_hbm.at[idx_v.at[jp]],
                    vals_v.at[:, pl.ds(jp * _CHUNK, _CHUNK)],
                    sem,
                ).wait()
            return 0

        lax.fori_loop(0, rows_per_w + _INFLIGHT, fire, 0, unroll=False)

        cpw = b_per_w // _CHUNK
        for g in range(groups):
            c = (g * _L) // _CHUNK
            t = (g * _L) % _CHUNK
            acc = jnp.zeros((_L,), jnp.float32)
            for k in range(F):
                acc = acc + vals_v[0, pl.ds((k * cpw + c) * _CHUNK + t, _L)]
            out_v[pl.ds(g * _L, _L)] = acc

        pltpu.sync_copy(out_v, out_hbm.at[pl.ds(base, b_per_w)])

    return lr_kernel


@functools.lru_cache(maxsize=None)
def _fm_call(B, F, D):
    BB = 1024

    def _fm_body(x_ref, o_ref):
        x = x_ref[...]
        x3 = x.reshape(F, D, BB)
        s = jnp.sum(x3, axis=0)
        ss = jnp.sum(s * s, axis=0)
        sq = jnp.sum(x * x, axis=0)
        o_ref[...] = 0.5 * (ss - sq)

    return pl.pallas_call(
        _fm_body,
        grid=(B // BB,),
        in_specs=[pl.BlockSpec((F * D, BB), lambda i: (0, i))],
        out_specs=pl.BlockSpec((BB,), lambda i: (i,)),
        out_shape=jax.ShapeDtypeStruct((B,), jnp.float32),
    )


def kernel(X, feature_emb, lr_table, bias):
    B, F = X.shape
    D = feature_emb.shape[2]
    V = lr_table.shape[0]

    bpw = B // _NW
    cpw = bpw // _CHUNK
    x2 = (X.T.reshape(F, _NW, cpw, _CHUNK)
          .transpose(1, 0, 2, 3)
          .reshape((B * F) // _CHUNK, 1, _CHUNK))
    lr = _lr_call(B, F, V)(x2, lr_table.T)

    xT = feature_emb.reshape(B, F * D).T
    fm = _fm_call(B, F, D)(xT)

    return (fm + lr + bias[0])[:, None]


# speedup vs baseline: 6.7072x; 1.0128x over previous
import functools

import jax
import jax.numpy as jnp
from jax import lax
from jax.experimental import pallas as pl
from jax.experimental.pallas import tpu as pltpu
from jax.experimental.pallas import tpu_sc as plsc

_NC = 2
_NS = 16
_NW = _NC * _NS
_L = 16
_CHUNK = 128
_INFLIGHT = 16


@functools.lru_cache(maxsize=None)
def _lr_call(B, F, V):
    n_per_w = (B * F) // _NW
    rows_per_w = n_per_w // _CHUNK
    b_per_w = B // _NW
    groups = b_per_w // _L
    cpw = b_per_w // _CHUNK

    mesh = plsc.VectorSubcoreMesh(core_axis_name="c", subcore_axis_name="s")

    @functools.partial(
        pl.kernel,
        out_type=jax.ShapeDtypeStruct((B,), jnp.float32),
        mesh=mesh,
        scratch_types=[
            pltpu.VMEM((F, b_per_w), jnp.int32),
            pltpu.VMEM((1, n_per_w), jnp.float32),
            pltpu.VMEM((b_per_w,), jnp.float32),
            pltpu.SemaphoreType.DMA,
        ],
    )
    def lr_kernel(xt_hbm, table_hbm, out_hbm, idx_v, vals_v, out_v, sem):
        wid = lax.axis_index("s") * _NC + lax.axis_index("c")
        base = wid * b_per_w

        pltpu.sync_copy(xt_hbm.at[:, pl.ds(base, b_per_w)], idx_v)

        def offsets(j):
            k = j // cpw
            c = lax.rem(j, cpw)
            return idx_v.at[pl.ds(k, 1), pl.ds(c * _CHUNK, _CHUNK)]

        def fire(j, _):
            @pl.when(j < rows_per_w)
            def _():
                pltpu.async_copy(
                    table_hbm.at[offsets(j)],
                    vals_v.at[:, pl.ds(j * _CHUNK, _CHUNK)],
                    sem,
                )

            @pl.when(j >= _INFLIGHT)
            def _():
                jp = j - _INFLIGHT
                pltpu.make_async_copy(
                    table_hbm.at[offsets(jp)],
                    vals_v.at[:, pl.ds(jp * _CHUNK, _CHUNK)],
                    sem,
                ).wait()
            return 0

        lax.fori_loop(0, rows_per_w + _INFLIGHT, fire, 0, unroll=False)

        for g in range(groups):
            c = (g * _L) // _CHUNK
            t = (g * _L) % _CHUNK
            acc = jnp.zeros((_L,), jnp.float32)
            for k in range(F):
                acc = acc + vals_v[0, pl.ds((k * cpw + c) * _CHUNK + t, _L)]
            out_v[pl.ds(g * _L, _L)] = acc

        pltpu.sync_copy(out_v, out_hbm.at[pl.ds(base, b_per_w)])

    return lr_kernel


@functools.lru_cache(maxsize=None)
def _fm_call(B, F, D):
    BB = 1024

    def _fm_body(x_ref, o_ref):
        x = x_ref[...]
        x3 = x.reshape(F, D, BB)
        s = jnp.sum(x3, axis=0)
        ss = jnp.sum(s * s, axis=0)
        sq = jnp.sum(x * x, axis=0)
        o_ref[...] = 0.5 * (ss - sq)

    return pl.pallas_call(
        _fm_body,
        grid=(B // BB,),
        in_specs=[pl.BlockSpec((F * D, BB), lambda i: (0, i))],
        out_specs=pl.BlockSpec((BB,), lambda i: (i,)),
        out_shape=jax.ShapeDtypeStruct((B,), jnp.float32),
    )


def kernel(X, feature_emb, lr_table, bias):
    B, F = X.shape
    D = feature_emb.shape[2]
    V = lr_table.shape[0]

    lr = _lr_call(B, F, V)(X.T, lr_table.T)

    xT = feature_emb.reshape(B, F * D).T
    fm = _fm_call(B, F, D)(xT)

    return (fm + lr + bias[0])[:, None]


# speedup vs baseline: 6.8597x; 1.0227x over previous
import functools

import jax
import jax.numpy as jnp
from jax import lax
from jax.experimental import pallas as pl
from jax.experimental.pallas import tpu as pltpu
from jax.experimental.pallas import tpu_sc as plsc

_NC = 2
_NS = 16
_NW = _NC * _NS
_L = 16
_CHUNK = 128
_INFLIGHT = 32


@functools.lru_cache(maxsize=None)
def _lr_call(B, F, V):
    n_per_w = (B * F) // _NW
    rows_per_w = n_per_w // _CHUNK
    b_per_w = B // _NW
    groups = b_per_w // _L
    cpw = b_per_w // _CHUNK

    mesh = plsc.VectorSubcoreMesh(core_axis_name="c", subcore_axis_name="s")

    @functools.partial(
        pl.kernel,
        out_type=jax.ShapeDtypeStruct((B,), jnp.float32),
        mesh=mesh,
        scratch_types=[
            pltpu.VMEM((F, b_per_w), jnp.int32),
            pltpu.VMEM((1, n_per_w), jnp.float32),
            pltpu.VMEM((b_per_w,), jnp.float32),
            pltpu.SemaphoreType.DMA,
        ],
    )
    def lr_kernel(xt_hbm, table_hbm, out_hbm, idx_v, vals_v, out_v, sem):
        wid = lax.axis_index("s") * _NC + lax.axis_index("c")
        base = wid * b_per_w

        pltpu.sync_copy(xt_hbm.at[:, pl.ds(base, b_per_w)], idx_v)

        def offsets(j):
            k = j // cpw
            c = lax.rem(j, cpw)
            return idx_v.at[pl.ds(k, 1), pl.ds(c * _CHUNK, _CHUNK)]

        def fire(j, _):
            @pl.when(j < rows_per_w)
            def _():
                pltpu.async_copy(
                    table_hbm.at[offsets(j)],
                    vals_v.at[:, pl.ds(j * _CHUNK, _CHUNK)],
                    sem,
                )

            @pl.when(j >= _INFLIGHT)
            def _():
                jp = j - _INFLIGHT
                pltpu.make_async_copy(
                    table_hbm.at[offsets(jp)],
                    vals_v.at[:, pl.ds(jp * _CHUNK, _CHUNK)],
                    sem,
                ).wait()
            return 0

        lax.fori_loop(0, rows_per_w + _INFLIGHT, fire, 0, unroll=4)

        for g in range(groups):
            c = (g * _L) // _CHUNK
            t = (g * _L) % _CHUNK
            acc = jnp.zeros((_L,), jnp.float32)
            for k in range(F):
                acc = acc + vals_v[0, pl.ds((k * cpw + c) * _CHUNK + t, _L)]
            out_v[pl.ds(g * _L, _L)] = acc

        pltpu.sync_copy(out_v, out_hbm.at[pl.ds(base, b_per_w)])

    return lr_kernel


@functools.lru_cache(maxsize=None)
def _fm_call(B, F, D):
    BB = 1024

    def _fm_body(x_ref, o_ref):
        x = x_ref[...]
        x3 = x.reshape(F, D, BB)
        s = jnp.sum(x3, axis=0)
        ss = jnp.sum(s * s, axis=0)
        sq = jnp.sum(x * x, axis=0)
        o_ref[...] = 0.5 * (ss - sq)

    return pl.pallas_call(
        _fm_body,
        grid=(B // BB,),
        in_specs=[pl.BlockSpec((F * D, BB), lambda i: (0, i))],
        out_specs=pl.BlockSpec((BB,), lambda i: (i,)),
        out_shape=jax.ShapeDtypeStruct((B,), jnp.float32),
    )


def kernel(X, feature_emb, lr_table, bias):
    B, F = X.shape
    D = feature_emb.shape[2]
    V = lr_table.shape[0]

    lr = _lr_call(B, F, V)(X.T, lr_table.T)

    xT = feature_emb.reshape(B, F * D).T
    fm = _fm_call(B, F, D)(xT)

    return (fm + lr + bias[0])[:, None]


# speedup vs baseline: 7.0827x; 1.0325x over previous
import functools

import jax
import jax.numpy as jnp
from jax import lax
from jax.experimental import pallas as pl
from jax.experimental.pallas import tpu as pltpu
from jax.experimental.pallas import tpu_sc as plsc

_NC = 2
_NS = 16
_NW = _NC * _NS
_L = 16
_CHUNK = 128
_INFLIGHT = 32


@functools.lru_cache(maxsize=None)
def _lr_call(B, F, V):
    n_per_w = (B * F) // _NW
    rows_per_w = n_per_w // _CHUNK
    b_per_w = B // _NW
    groups = b_per_w // _L
    cpw = b_per_w // _CHUNK

    mesh = plsc.VectorSubcoreMesh(core_axis_name="c", subcore_axis_name="s")

    @functools.partial(
        pl.kernel,
        out_type=jax.ShapeDtypeStruct((B,), jnp.float32),
        mesh=mesh,
        scratch_types=[
            pltpu.VMEM((F, b_per_w), jnp.int32),
            pltpu.VMEM((1, n_per_w), jnp.float32),
            pltpu.VMEM((b_per_w,), jnp.float32),
            pltpu.SemaphoreType.DMA,
        ],
    )
    def lr_kernel(xt_hbm, table_hbm, out_hbm, idx_v, vals_v, out_v, sem):
        wid = lax.axis_index("s") * _NC + lax.axis_index("c")
        base = wid * b_per_w

        pltpu.sync_copy(xt_hbm.at[:, pl.ds(base, b_per_w)], idx_v)

        def offsets(j):
            k = j // cpw
            c = lax.rem(j, cpw)
            return idx_v.at[pl.ds(k, 1), pl.ds(c * _CHUNK, _CHUNK)]

        def fire(j, _):
            @pl.when(j < rows_per_w)
            def _():
                pltpu.async_copy(
                    table_hbm.at[offsets(j)],
                    vals_v.at[:, pl.ds(j * _CHUNK, _CHUNK)],
                    sem,
                )

            @pl.when(j >= _INFLIGHT)
            def _():
                jp = j - _INFLIGHT
                pltpu.make_async_copy(
                    table_hbm.at[offsets(jp)],
                    vals_v.at[:, pl.ds(jp * _CHUNK, _CHUNK)],
                    sem,
                ).wait()
            return 0

        lax.fori_loop(0, rows_per_w + _INFLIGHT, fire, 0, unroll=4)

        def seg(g, _):
            acc = jnp.zeros((_L,), jnp.float32)
            for k in range(F):
                acc = acc + vals_v[0, pl.ds(k * b_per_w + g * _L, _L)]
            out_v[pl.ds(g * _L, _L)] = acc
            return 0

        lax.fori_loop(0, groups, seg, 0, unroll=2)

        pltpu.sync_copy(out_v, out_hbm.at[pl.ds(base, b_per_w)])

    return lr_kernel


@functools.lru_cache(maxsize=None)
def _fm_call(B, F, D):
    BB = 1024

    def _fm_body(x_ref, o_ref):
        x = x_ref[...]
        x3 = x.reshape(F, D, BB)
        s = jnp.sum(x3, axis=0)
        ss = jnp.sum(s * s, axis=0)
        sq = jnp.sum(x * x, axis=0)
        o_ref[...] = 0.5 * (ss - sq)

    return pl.pallas_call(
        _fm_body,
        grid=(B // BB,),
        in_specs=[pl.BlockSpec((F * D, BB), lambda i: (0, i))],
        out_specs=pl.BlockSpec((BB,), lambda i: (i,)),
        out_shape=jax.ShapeDtypeStruct((B,), jnp.float32),
    )


def kernel(X, feature_emb, lr_table, bias):
    B, F = X.shape
    D = feature_emb.shape[2]
    V = lr_table.shape[0]

    lr = _lr_call(B, F, V)(X.T, lr_table.T)

    xT = feature_emb.reshape(B, F * D).T
    fm = _fm_call(B, F, D)(xT)

    return (fm + lr + bias[0])[:, None]


# speedup vs baseline: 7.5226x; 1.0621x over previous
import functools

import jax
import jax.numpy as jnp
from jax import lax
from jax.experimental import pallas as pl
from jax.experimental.pallas import tpu as pltpu
from jax.experimental.pallas import tpu_sc as plsc

_NC = 2
_NS = 16
_NW = _NC * _NS
_L = 16
_CHUNK = 128
_INFLIGHT = 32


@functools.lru_cache(maxsize=None)
def _lr_call(B, F, V):
    n_per_w = (B * F) // _NW
    rows_per_w = n_per_w // _CHUNK
    b_per_w = B // _NW
    groups = b_per_w // _L
    cpw = b_per_w // _CHUNK

    mesh = plsc.VectorSubcoreMesh(core_axis_name="c", subcore_axis_name="s")

    @functools.partial(
        pl.kernel,
        out_type=jax.ShapeDtypeStruct((B,), jnp.float32),
        mesh=mesh,
        scratch_types=[
            pltpu.VMEM((F, b_per_w), jnp.int32),
            pltpu.VMEM((1, n_per_w), jnp.float32),
            pltpu.VMEM((b_per_w,), jnp.float32),
            pltpu.SemaphoreType.DMA,
            pltpu.SemaphoreType.DMA,
        ],
    )
    def lr_kernel(xt_hbm, table_hbm, out_hbm, idx_v, vals_v, out_v,
                  sem_a, sem_b):
        wid = lax.axis_index("s") * _NC + lax.axis_index("c")
        base = wid * b_per_w
        sems = (sem_a, sem_b)
        chunk_bytes_cols = F * _CHUNK

        def stage(c):
            pltpu.sync_copy(
                xt_hbm.at[:, pl.ds(base + c * _CHUNK, _CHUNK)],
                idx_v.at[:, pl.ds(c * _CHUNK, _CHUNK)],
            )

        def fire_chunk(c, sem):
            def fk(k, _):
                pltpu.async_copy(
                    table_hbm.at[idx_v.at[pl.ds(k, 1),
                                          pl.ds(c * _CHUNK, _CHUNK)]],
                    vals_v.at[:, pl.ds(k * b_per_w + c * _CHUNK, _CHUNK)],
                    sem,
                )
                return 0
            lax.fori_loop(0, F, fk, 0, unroll=2)

        def drain(sem):
            pltpu.make_async_copy(
                table_hbm.at[:, pl.ds(0, chunk_bytes_cols)],
                vals_v.at[:, pl.ds(0, chunk_bytes_cols)],
                sem,
            ).wait()

        def compute(c):
            def seg(gl, _):
                g = c * (_CHUNK // _L) + gl
                acc = jnp.zeros((_L,), jnp.float32)
                for k in range(F):
                    acc = acc + vals_v[0, pl.ds(k * b_per_w + g * _L, _L)]
                out_v[pl.ds(g * _L, _L)] = acc
                return 0
            lax.fori_loop(0, _CHUNK // _L, seg, 0, unroll=2)

        stage(0)
        fire_chunk(0, sems[0])
        stage(1)
        fire_chunk(1, sems[1])
        for c in range(cpw):
            sem = sems[c % 2]
            drain(sem)
            if c + 2 < cpw:
                stage(c + 2)
                fire_chunk(c + 2, sem)
            compute(c)

        pltpu.sync_copy(out_v, out_hbm.at[pl.ds(base, b_per_w)])

    return lr_kernel


@functools.lru_cache(maxsize=None)
def _fm_call(B, F, D):
    BB = 1024

    def _fm_body(x_ref, o_ref):
        x = x_ref[...]
        x3 = x.reshape(F, D, BB)
        s = jnp.sum(x3, axis=0)
        ss = jnp.sum(s * s, axis=0)
        sq = jnp.sum(x * x, axis=0)
        o_ref[...] = 0.5 * (ss - sq)

    return pl.pallas_call(
        _fm_body,
        grid=(B // BB,),
        in_specs=[pl.BlockSpec((F * D, BB), lambda i: (0, i))],
        out_specs=pl.BlockSpec((BB,), lambda i: (i,)),
        out_shape=jax.ShapeDtypeStruct((B,), jnp.float32),
    )


def kernel(X, feature_emb, lr_table, bias):
    B, F = X.shape
    D = feature_emb.shape[2]
    V = lr_table.shape[0]

    lr = _lr_call(B, F, V)(X.T, lr_table.T)

    xT = feature_emb.reshape(B, F * D).T
    fm = _fm_call(B, F, D)(xT)

    return (fm + lr + bias[0])[:, None]
